# Initial kernel scaffold; baseline (speedup 1.0000x reference)
import functools

import jax
import jax.numpy as jnp
from jax import lax
from jax.experimental import pallas as pl
from jax.experimental.pallas import tpu as pltpu
from jax.experimental.pallas import tpu_sc as plsc

_N = 10000
_E = 320000
_NC = 2
_NS = 16
_NW = _NC * _NS
_C = 128
_K = 79
_EPAD = _NW * _K * _C
_NPAD = 10016
_RPS = _NPAD // _NS


def _prep_edges(edge_index):
    src = edge_index[0]
    dst = edge_index[1]
    pad = _EPAD - _E
    src = jnp.concatenate([src, jnp.zeros((pad,), jnp.int32)])
    dst = jnp.concatenate([dst, jnp.full((pad,), _N, jnp.int32)])
    return src.reshape(_NW, _K, _C), dst.reshape(_NW, _K, _C)


def _sc_mesh():
    return plsc.VectorSubcoreMesh(core_axis_name="c", subcore_axis_name="s")


def _sc_hist(dst_w, ones_hbm, zrows_hbm):
    @functools.partial(
        pl.kernel,
        out_type=jax.ShapeDtypeStruct((_NC, _NPAD, 16), jnp.float32),
        mesh=_sc_mesh(),
        scratch_types=[
            pltpu.VMEM((_K, _C), jnp.int32),
            pltpu.VMEM((_C, 16), jnp.float32),
            pltpu.VMEM_SHARED((_NPAD, 16), jnp.float32),
        ],
    )
    def hist_kernel(dst_hbm, ones_h, z_h, out_hbm, idx_v, ones_v, acc):
        cid = lax.axis_index("c")
        sid = lax.axis_index("s")
        wid = cid * _NS + sid
        base = sid * _RPS
        pltpu.sync_copy(z_h, acc.at[pl.ds(base, _RPS)])
        pltpu.sync_copy(ones_h, ones_v)
        pltpu.sync_copy(dst_hbm.at[wid], idx_v)
        plsc.subcore_barrier()

        @pl.loop(0, _K)
        def _(j):
            pltpu.sync_copy(ones_v, acc.at[idx_v.at[j]], add=True)

        plsc.subcore_barrier()
        pltpu.sync_copy(acc.at[pl.ds(base, _RPS)],
                        out_hbm.at[cid].at[pl.ds(base, _RPS)])

    return hist_kernel(dst_w, ones_hbm, zrows_hbm)


def _sc_scatter(y, src_w, dst_w, zrows_hbm, d):
    @functools.partial(
        pl.kernel,
        out_type=jax.ShapeDtypeStruct((_NC, _NPAD, d), jnp.float32),
        mesh=_sc_mesh(),
        scratch_types=[
            pltpu.VMEM((_K, _C), jnp.int32),
            pltpu.VMEM((_K, _C), jnp.int32),
            pltpu.VMEM((_C, d), jnp.float32),
            pltpu.VMEM_SHARED((_NPAD, d), jnp.float32),
            pltpu.SemaphoreType.DMA,
        ],
    )
    def scatter_kernel(y_hbm, src_hbm, dst_hbm, z_h, out_hbm,
                       idx_s, idx_d, buf, acc, sem):
        cid = lax.axis_index("c")
        sid = lax.axis_index("s")
        wid = cid * _NS + sid
        base = sid * _RPS
        pltpu.sync_copy(z_h, acc.at[pl.ds(base, _RPS)])
        pltpu.sync_copy(src_hbm.at[wid], idx_s)
        pltpu.sync_copy(dst_hbm.at[wid], idx_d)
        plsc.subcore_barrier()

        @pl.loop(0, _K)
        def _(j):
            pltpu.async_copy(y_hbm.at[idx_s.at[j]], buf, sem).wait()
            pltpu.sync_copy(buf, acc.at[idx_d.at[j]], add=True)

        plsc.subcore_barrier()
        pltpu.sync_copy(acc.at[pl.ds(base, _RPS)],
                        out_hbm.at[cid].at[pl.ds(base, _RPS)])

    return scatter_kernel(y, src_w, dst_w, zrows_hbm)


def _dinv_from_hist(hist_ref):
    deg = hist_ref[0, :_N, 0] + hist_ref[1, :_N, 0] + 1.0
    return lax.rsqrt(deg)


def _tc_layer_in(x, w, hist):
    def body(x_ref, w_ref, h_ref, y_ref):
        dinv = _dinv_from_hist(h_ref)
        xw = jnp.dot(x_ref[...], w_ref[...],
                     preferred_element_type=jnp.float32)
        y_ref[...] = xw * dinv[:, None]

    return pl.pallas_call(
        body,
        out_shape=jax.ShapeDtypeStruct((_N, w.shape[1]), jnp.float32),
    )(x, w, hist)


def _tc_layer_mid(s, y0, hist, w, b):
    def body(s_ref, y0_ref, h_ref, w_ref, b_ref, y_ref):
        dinv = _dinv_from_hist(h_ref)
        agg = s_ref[0, :_N, :] + s_ref[1, :_N, :] + y0_ref[...]
        hid = jnp.maximum(agg * dinv[:, None] + b_ref[...][None, :], 0.0)
        y_ref[...] = jnp.dot(hid, w_ref[...],
                             preferred_element_type=jnp.float32) * dinv[:, None]

    return pl.pallas_call(
        body,
        out_shape=jax.ShapeDtypeStruct((_N, w.shape[1]), jnp.float32),
    )(s, y0, hist, w, b)


def _tc_layer_out(s, y1, hist, b):
    def body(s_ref, y1_ref, h_ref, b_ref, z_ref):
        dinv = _dinv_from_hist(h_ref)
        agg = s_ref[0, :_N, :] + s_ref[1, :_N, :] + y1_ref[...]
        z_ref[...] = jnp.maximum(agg * dinv[:, None] + b_ref[...][None, :], 0.0)

    return pl.pallas_call(
        body,
        out_shape=jax.ShapeDtypeStruct((_N, y1.shape[1]), jnp.float32),
    )(s, y1, hist, b)


def kernel(x_a, edge_a, x_b, edge_b, W0, b0, W1, b1):
    src_a, dst_a = _prep_edges(edge_a)
    src_b, dst_b = _prep_edges(edge_b)
    ones16 = jnp.ones((_C, 16), jnp.float32)
    z16 = jnp.zeros((_RPS, 16), jnp.float32)
    z128 = jnp.zeros((_RPS, 128), jnp.float32)
    z64 = jnp.zeros((_RPS, 64), jnp.float32)

    hist_a = _sc_hist(dst_a, ones16, z16)
    hist_b = _sc_hist(dst_b, ones16, z16)

    y0a = _tc_layer_in(x_a, W0, hist_a)
    s0a = _sc_scatter(y0a, src_a, dst_a, z128, 128)
    y0b = _tc_layer_in(x_b, W0, hist_b)
    s0b = _sc_scatter(y0b, src_b, dst_b, z128, 128)

    y1a = _tc_layer_mid(s0a, y0a, hist_a, W1, b0)
    s1a = _sc_scatter(y1a, src_a, dst_a, z64, 64)
    y1b = _tc_layer_mid(s0b, y0b, hist_b, W1, b0)
    s1b = _sc_scatter(y1b, src_b, dst_b, z64, 64)

    z_a = _tc_layer_out(s1a, y1a, hist_a, b1)
    z_b = _tc_layer_out(s1b, y1b, hist_b, b1)
    return (z_a, z_b)


# speedup vs baseline: 10.2100x; 10.2100x over previous
import dataclasses
import functools

import jax
import jax.numpy as jnp
from jax import lax
from jax.experimental import pallas as pl
from jax.experimental.pallas import tpu as pltpu
from jax.experimental.pallas import tpu_sc as plsc

_N = 10000
_E = 320000
_NC = 2
_NS = 16
_NW = _NC * _NS
_C = 128
_K = 79
_EPAD = _NW * _K * _C
_NPAD = 10112
_RPS = _NPAD // _NS


def _prep_edges(edge_index):
    src = edge_index[0]
    dst = edge_index[1]
    pad = _EPAD - _E
    src = jnp.concatenate([src, jnp.zeros((pad,), jnp.int32)])
    dst = jnp.concatenate([dst, jnp.full((pad,), _N, jnp.int32)])
    return src.reshape(_NW, _K, _C), dst.reshape(_NW, _K, _C)


def _sc_mesh():
    return plsc.VectorSubcoreMesh(core_axis_name="c", subcore_axis_name="s")


_PW = _K * _C


def _sc_hist(dst_w, z_hbm):
    @functools.partial(
        pl.kernel,
        out_type=jax.ShapeDtypeStruct((_NW, _NPAD), jnp.float32),
        mesh=_sc_mesh(),
        scratch_types=[
            pltpu.VMEM((_PW,), jnp.int32),
            pltpu.VMEM((_NPAD,), jnp.float32),
        ],
        compiler_params=dataclasses.replace(
            pltpu.CompilerParams(), needs_layout_passes=False),
    )
    def hist_kernel(dst_hbm, z_h, out_hbm, idx_v, acc):
        cid = lax.axis_index("c")
        sid = lax.axis_index("s")
        wid = cid * _NS + sid
        pltpu.sync_copy(z_h, acc)
        pltpu.sync_copy(dst_hbm.at[wid], idx_v)
        ones = jnp.ones((16,), jnp.float32)

        @pl.loop(0, _PW // 16)
        def _(i):
            iv = idx_v[pl.ds(i * 16, 16)]
            plsc.addupdate_scatter(acc, [iv], ones)

        pltpu.sync_copy(acc, out_hbm.at[wid])

    return hist_kernel(dst_w.reshape(_NW, _PW), z_hbm)


def _sc_scatter(y, src_w, dst_w, zrows_hbm, d):
    @functools.partial(
        pl.kernel,
        out_type=jax.ShapeDtypeStruct((_NC, _NPAD, d), jnp.float32),
        mesh=_sc_mesh(),
        scratch_types=[
            pltpu.VMEM((_K, _C), jnp.int32),
            pltpu.VMEM((_K, _C), jnp.int32),
            pltpu.VMEM((_C, d), jnp.float32),
            pltpu.VMEM_SHARED((_NPAD, d), jnp.float32),
            pltpu.SemaphoreType.DMA,
        ],
    )
    def scatter_kernel(y_hbm, src_hbm, dst_hbm, z_h, out_hbm,
                       idx_s, idx_d, buf, acc, sem):
        cid = lax.axis_index("c")
        sid = lax.axis_index("s")
        wid = cid * _NS + sid
        base = sid * _RPS
        pltpu.sync_copy(z_h, acc.at[pl.ds(base, _RPS)])
        pltpu.sync_copy(src_hbm.at[wid], idx_s)
        pltpu.sync_copy(dst_hbm.at[wid], idx_d)
        plsc.subcore_barrier()

        @pl.loop(0, _K)
        def _(j):
            pltpu.async_copy(y_hbm.at[idx_s.at[j]], buf, sem).wait()
            pltpu.sync_copy(buf, acc.at[idx_d.at[j]], add=True)

        plsc.subcore_barrier()
        pltpu.sync_copy(acc.at[pl.ds(base, _RPS)],
                        out_hbm.at[cid].at[pl.ds(base, _RPS)])

    return scatter_kernel(y, src_w, dst_w, zrows_hbm)


def _dinv_from_hist(hist_ref):
    deg = jnp.sum(hist_ref[:, :_N], axis=0) + 1.0
    return lax.rsqrt(deg)


def _tc_layer_in(x, w, hist):
    def body(x_ref, w_ref, h_ref, y_ref):
        dinv = _dinv_from_hist(h_ref)
        xw = jnp.dot(x_ref[...], w_ref[...],
                     preferred_element_type=jnp.float32)
        y_ref[...] = xw * dinv[:, None]

    return pl.pallas_call(
        body,
        out_shape=jax.ShapeDtypeStruct((_N, w.shape[1]), jnp.float32),
    )(x, w, hist)


def _tc_layer_mid(s, y0, hist, w, b):
    def body(s_ref, y0_ref, h_ref, w_ref, b_ref, y_ref):
        dinv = _dinv_from_hist(h_ref)
        agg = s_ref[0, :_N, :] + s_ref[1, :_N, :] + y0_ref[...]
        hid = jnp.maximum(agg * dinv[:, None] + b_ref[...][None, :], 0.0)
        y_ref[...] = jnp.dot(hid, w_ref[...],
                             preferred_element_type=jnp.float32) * dinv[:, None]

    return pl.pallas_call(
        body,
        out_shape=jax.ShapeDtypeStruct((_N, w.shape[1]), jnp.float32),
    )(s, y0, hist, w, b)


def _tc_layer_out(s, y1, hist, b, d_out):
    def body(s_ref, y1_ref, h_ref, b_ref, z_ref):
        dinv = _dinv_from_hist(h_ref)
        agg = (s_ref[0, :_N, :d_out] + s_ref[1, :_N, :d_out]
               + y1_ref[:, :d_out])
        z_ref[...] = jnp.maximum(agg * dinv[:, None] + b_ref[...][None, :], 0.0)

    return pl.pallas_call(
        body,
        out_shape=jax.ShapeDtypeStruct((_N, d_out), jnp.float32),
    )(s, y1, hist, b)


def kernel(x_a, edge_a, x_b, edge_b, W0, b0, W1, b1):
    src_a, dst_a = _prep_edges(edge_a)
    src_b, dst_b = _prep_edges(edge_b)
    zN = jnp.zeros((_NPAD,), jnp.float32)
    z128 = jnp.zeros((_RPS, 128), jnp.float32)
    W1p = jnp.pad(W1, ((0, 0), (0, 128 - W1.shape[1])))

    hist_a = _sc_hist(dst_a, zN)
    hist_b = _sc_hist(dst_b, zN)

    y0a = _tc_layer_in(x_a, W0, hist_a)
    s0a = _sc_scatter(y0a, src_a, dst_a, z128, 128)
    y0b = _tc_layer_in(x_b, W0, hist_b)
    s0b = _sc_scatter(y0b, src_b, dst_b, z128, 128)

    y1a = _tc_layer_mid(s0a, y0a, hist_a, W1p, b0)
    s1a = _sc_scatter(y1a, src_a, dst_a, z128, 128)
    y1b = _tc_layer_mid(s0b, y0b, hist_b, W1p, b0)
    s1b = _sc_scatter(y1b, src_b, dst_b, z128, 128)

    z_a = _tc_layer_out(s1a, y1a, hist_a, b1, 64)
    z_b = _tc_layer_out(s1b, y1b, hist_b, b1, 64)
    return (z_a, z_b)
